# baseline (speedup 1.0000x reference)
import functools

import jax
import jax.numpy as jnp
from jax import lax
from jax.experimental import pallas as pl
from jax.experimental.pallas import tpu as pltpu
from jax.experimental.pallas import tpu_sc as plsc

ROWS = 8192
COLS = 1024
NUM_CORES = 2
NUM_SUBCORES = 16
NUM_WORKERS = NUM_CORES * NUM_SUBCORES
ROWS_PER_WORKER = ROWS // NUM_WORKERS

_MESH = plsc.VectorSubcoreMesh(
    core_axis_name="c", subcore_axis_name="s", num_cores=NUM_CORES
)


@functools.partial(
    pl.kernel,
    mesh=_MESH,
    out_type=jax.ShapeDtypeStruct((ROWS, COLS), jnp.float32),
)
def _copy_sc(pos_hbm, out_hbm):
    wid = lax.axis_index("s") * NUM_CORES + lax.axis_index("c")
    base = wid * ROWS_PER_WORKER
    pltpu.sync_copy(
        pos_hbm.at[pl.ds(base, ROWS_PER_WORKER)],
        out_hbm.at[pl.ds(base, ROWS_PER_WORKER)],
    )


def kernel(x, pos_emb):
    del x
    return _copy_sc(pos_emb)


# speedup vs baseline: 23.0538x; 23.0538x over previous
import functools

import jax
import jax.numpy as jnp
from jax import lax
from jax.experimental import pallas as pl
from jax.experimental.pallas import tpu as pltpu
from jax.experimental.pallas import tpu_sc as plsc

ROWS = 8192
COLS = 1024
NUM_CORES = 2
NUM_SUBCORES = 16
NUM_WORKERS = NUM_CORES * NUM_SUBCORES
ROWS_PER_WORKER = ROWS // NUM_WORKERS
CHUNK = 32
NCHUNK = ROWS_PER_WORKER // CHUNK

_MESH = plsc.VectorSubcoreMesh(
    core_axis_name="c", subcore_axis_name="s", num_cores=NUM_CORES
)


@functools.partial(
    pl.kernel,
    mesh=_MESH,
    out_type=jax.ShapeDtypeStruct((ROWS, COLS), jnp.float32),
    scratch_types=[
        pltpu.VMEM((CHUNK, COLS), jnp.float32),
        pltpu.VMEM((CHUNK, COLS), jnp.float32),
        pltpu.SemaphoreType.DMA,
        pltpu.SemaphoreType.DMA,
        pltpu.SemaphoreType.DMA,
        pltpu.SemaphoreType.DMA,
    ],
)
def _copy_sc(pos_hbm, out_hbm, buf0, buf1, rs0, rs1, ws0, ws1):
    wid = lax.axis_index("s") * NUM_CORES + lax.axis_index("c")
    base = wid * ROWS_PER_WORKER
    bufs = (buf0, buf1)
    rsems = (rs0, rs1)
    wsems = (ws0, ws1)

    def read(i):
        return pltpu.make_async_copy(
            pos_hbm.at[pl.ds(base + i * CHUNK, CHUNK)], bufs[i % 2], rsems[i % 2]
        )

    def write(i):
        return pltpu.make_async_copy(
            bufs[i % 2], out_hbm.at[pl.ds(base + i * CHUNK, CHUNK)], wsems[i % 2]
        )

    read(0).start()
    for i in range(NCHUNK):
        read(i).wait()
        if i + 1 < NCHUNK:
            if i >= 1:
                write(i - 1).wait()
            read(i + 1).start()
        write(i).start()
    write(NCHUNK - 2).wait()
    write(NCHUNK - 1).wait()


def kernel(x, pos_emb):
    del x
    return _copy_sc(pos_emb)


# speedup vs baseline: 24.8646x; 1.0785x over previous
import functools

import jax
import jax.numpy as jnp
from jax import lax
from jax.experimental import pallas as pl
from jax.experimental.pallas import tpu as pltpu
from jax.experimental.pallas import tpu_sc as plsc

ROWS = 8192
COLS = 1024
NUM_CORES = 2
NUM_SUBCORES = 16
NUM_WORKERS = NUM_CORES * NUM_SUBCORES
ROWS_PER_WORKER = ROWS // NUM_WORKERS
CHUNK = 32
NCHUNK = ROWS_PER_WORKER // CHUNK

_MESH = plsc.VectorSubcoreMesh(
    core_axis_name="c", subcore_axis_name="s", num_cores=NUM_CORES
)


@functools.partial(
    pl.kernel,
    mesh=_MESH,
    out_type=jax.ShapeDtypeStruct((ROWS, COLS), jnp.float32),
    scratch_types=[
        pltpu.VMEM((CHUNK, COLS), jnp.float32),
        pltpu.VMEM((CHUNK, COLS), jnp.float32),
        pltpu.VMEM((CHUNK, COLS), jnp.float32),
        pltpu.SemaphoreType.DMA,
        pltpu.SemaphoreType.DMA,
        pltpu.SemaphoreType.DMA,
        pltpu.SemaphoreType.DMA,
        pltpu.SemaphoreType.DMA,
        pltpu.SemaphoreType.DMA,
    ],
)
def _copy_sc(pos_hbm, out_hbm, buf0, buf1, buf2, rs0, rs1, rs2, ws0, ws1, ws2):
    wid = lax.axis_index("s") * NUM_CORES + lax.axis_index("c")
    base = wid * ROWS_PER_WORKER
    bufs = (buf0, buf1, buf2)
    rsems = (rs0, rs1, rs2)
    wsems = (ws0, ws1, ws2)
    nb = 3

    def read(i):
        return pltpu.make_async_copy(
            pos_hbm.at[pl.ds(base + i * CHUNK, CHUNK)], bufs[i % nb], rsems[i % nb]
        )

    def write(i):
        return pltpu.make_async_copy(
            bufs[i % nb], out_hbm.at[pl.ds(base + i * CHUNK, CHUNK)], wsems[i % nb]
        )

    read(0).start()
    read(1).start()
    for i in range(NCHUNK):
        read(i).wait()
        if i + 2 < NCHUNK:
            if i >= 1:
                write(i - 1).wait()
            read(i + 2).start()
        write(i).start()
    for j in range(max(0, NCHUNK - 2), NCHUNK):
        write(j).wait()
    if NCHUNK >= 3:
        write(NCHUNK - 3).wait()


def kernel(x, pos_emb):
    del x
    return _copy_sc(pos_emb)
